# baseline (speedup 1.0000x reference)
import functools

import jax
import jax.numpy as jnp
from jax.experimental import pallas as pl
from jax.experimental.pallas import tpu as pltpu


def _prep_kernel(x_ref, w_ref, b_ref, o_ref):
    o_ref[...] = (
        jnp.dot(x_ref[...], w_ref[...], preferred_element_type=jnp.float32)
        + b_ref[...]
    )


def _layer1_kernel(adj_ref, y1_ref, w2_ref, b2_ref, y2_ref):
    h = jnp.dot(adj_ref[...], y1_ref[...], preferred_element_type=jnp.float32)
    h = jnp.maximum(h, 0.0)
    y2_ref[...] = (
        jnp.dot(h, w2_ref[...], preferred_element_type=jnp.float32) + b2_ref[...]
    )


def _layer2_kernel(adj_ref, y2_ref, w3_ref, b3_ref, w4_ref, b4_ref,
                   emb_ref, z_ref):
    emb = jnp.dot(adj_ref[...], y2_ref[...], preferred_element_type=jnp.float32)
    emb_ref[...] = emb
    t = jnp.maximum(
        jnp.dot(emb, w3_ref[...], preferred_element_type=jnp.float32)
        + b3_ref[...],
        0.0,
    )
    z_ref[...] = (
        jnp.dot(t, w4_ref[...], preferred_element_type=jnp.float32) + b4_ref[...]
    )


def _pick_bm(n, target=400):
    best = None
    for bm in range(8, min(n, target) + 1, 8):
        if n % bm == 0:
            best = bm
    return best if best is not None else n


@jax.jit
def kernel(x, Adj_, W1, b1, W2, b2, W3, b3, W4, b4):
    n, in_dim = x.shape
    hid = W1.shape[1]
    emb_d = W2.shape[1]
    proj = W4.shape[1]
    f32 = jnp.float32

    b1r = b1.reshape(1, -1)
    b2r = b2.reshape(1, -1)
    b3r = b3.reshape(1, -1)
    b4r = b4.reshape(1, -1)

    y1 = pl.pallas_call(
        _prep_kernel,
        out_shape=jax.ShapeDtypeStruct((n, hid), f32),
    )(x, W1, b1r)

    bm = _pick_bm(n)
    grid = (n // bm,)

    adj_spec = pl.BlockSpec((bm, n), lambda i: (i, 0))
    full_rhs = lambda d: pl.BlockSpec((n, d), lambda i: (0, 0))
    small = lambda r, c: pl.BlockSpec((r, c), lambda i: (0, 0))
    row_out = lambda d: pl.BlockSpec((bm, d), lambda i: (i, 0))

    y2 = pl.pallas_call(
        _layer1_kernel,
        grid=grid,
        in_specs=[
            adj_spec,
            full_rhs(hid),
            small(hid, emb_d),
            small(1, emb_d),
        ],
        out_specs=row_out(emb_d),
        out_shape=jax.ShapeDtypeStruct((n, emb_d), f32),
        compiler_params=pltpu.CompilerParams(
            dimension_semantics=("arbitrary",),
        ),
    )(Adj_, y1, W2, b2r)

    emb, z = pl.pallas_call(
        _layer2_kernel,
        grid=grid,
        in_specs=[
            adj_spec,
            full_rhs(emb_d),
            small(emb_d, proj),
            small(1, proj),
            small(proj, proj),
            small(1, proj),
        ],
        out_specs=[row_out(emb_d), row_out(proj)],
        out_shape=[
            jax.ShapeDtypeStruct((n, emb_d), f32),
            jax.ShapeDtypeStruct((n, proj), f32),
        ],
        compiler_params=pltpu.CompilerParams(
            dimension_semantics=("arbitrary",),
        ),
    )(Adj_, y2, W3, b3r, W4, b4r)

    return (z, emb)
